# baseline (speedup 1.0000x reference)
import functools

import jax
import jax.numpy as jnp
from jax import lax
from jax.experimental import pallas as pl
from jax.experimental.pallas import tpu as pltpu
from jax.experimental.pallas import tpu_sc as plsc

N_NODES = 10000
N_EDGES = 320000
NUM_REL = 8
DIM = 128

NC = 2
NS = 16
NW = NC * NS

EROW = 64
E_PAD = 327680
IDX_ROWS = E_PAD // EROW
ROWS_PER_W = IDX_ROWS // NW
AGG_ROWS = 10112
ROWS_PER_TILE = AGG_ROWS // NS

N_BLK = 2000
N_GRID = N_NODES // N_BLK


def _dense_pre_body(h_ref, basis_ref, loop_ref, bias_ref, pck_ref, hsl_ref):
  hb0 = jnp.dot(h_ref[...], basis_ref[0], preferred_element_type=jnp.float32,
                precision=lax.Precision.HIGHEST)
  hb1 = jnp.dot(h_ref[...], basis_ref[1], preferred_element_type=jnp.float32,
                precision=lax.Precision.HIGHEST)

  def rnd(x):
    u = lax.bitcast_convert_type(x, jnp.uint32)
    return (u + jnp.uint32(0x7FFF) + ((u >> 16) & jnp.uint32(1))) >> 16

  pck_ref[...] = lax.bitcast_convert_type(rnd(hb0) | (rnd(hb1) << 16),
                                          jnp.int32)
  hsl_ref[...] = (
      jnp.dot(h_ref[...], loop_ref[...], preferred_element_type=jnp.float32,
              precision=lax.Precision.HIGHEST)
      + bias_ref[...])


def _dense_pre(h, basis, loop_w, bias2d):
  return pl.pallas_call(
      _dense_pre_body,
      grid=(N_GRID,),
      in_specs=[
          pl.BlockSpec((N_BLK, DIM), lambda n: (n, 0)),
          pl.BlockSpec((2, DIM, DIM), lambda n: (0, 0, 0)),
          pl.BlockSpec((DIM, DIM), lambda n: (0, 0)),
          pl.BlockSpec((1, DIM), lambda n: (0, 0)),
      ],
      out_specs=[
          pl.BlockSpec((N_BLK, DIM), lambda n: (n, 0)),
          pl.BlockSpec((N_BLK, DIM), lambda n: (n, 0)),
      ],
      out_shape=[
          jax.ShapeDtypeStruct((N_NODES, DIM), jnp.int32),
          jax.ShapeDtypeStruct((N_NODES, DIM), jnp.float32),
      ],
  )(h, basis, loop_w, bias2d)


def _wsel_body(et_ref, wcomp_ref, w0_ref, w1_ref):
  et = et_ref[...]
  wc = wcomp_ref[...]
  w0 = jnp.zeros_like(w0_ref)
  w1 = jnp.zeros_like(w1_ref)
  for r in range(NUM_REL):
    sel = et == r
    w0 = jnp.where(sel, wc[r, 0], w0)
    w1 = jnp.where(sel, wc[r, 1], w1)
  w0_ref[...] = w0
  w1_ref[...] = w1


def _wsel(et2d, wcomp):
  rows = E_PAD // DIM
  blk = rows // 10
  return pl.pallas_call(
      _wsel_body,
      grid=(10,),
      in_specs=[
          pl.BlockSpec((blk, DIM), lambda i: (i, 0)),
          pl.BlockSpec((NUM_REL, 2), lambda i: (0, 0)),
      ],
      out_specs=[pl.BlockSpec((blk, DIM), lambda i: (i, 0))] * 2,
      out_shape=[jax.ShapeDtypeStruct((rows, DIM), jnp.float32)] * 2,
  )(et2d, wcomp)


def _combine_body(act, p0_ref, p1_ref, hsl_ref, out_ref):
  out_ref[...] = act(p0_ref[...] + p1_ref[...] + hsl_ref[...])


def _combine(p0, p1, hsl, act):
  return pl.pallas_call(
      functools.partial(_combine_body, act),
      grid=(N_GRID,),
      in_specs=[pl.BlockSpec((N_BLK, DIM), lambda n: (n, 0))] * 3,
      out_specs=pl.BlockSpec((N_BLK, DIM), lambda n: (n, 0)),
      out_shape=jax.ShapeDtypeStruct((N_NODES, DIM), jnp.float32),
  )(p0, p1, hsl)


MCHUNK = 16
N_MCHUNK = ROWS_PER_W // MCHUNK


def _sc_body(pck_hbm, src_hbm, dst_hbm, w01_hbm, out_hbm,
             mbufs, wv, bbufs, fbufs, msem, gsems, ssems, agg_sh):
  c = lax.axis_index("c")
  s = lax.axis_index("s")
  wid = s * NC + c
  base = wid * ROWS_PER_W

  zero = jnp.zeros((16,), jnp.float32)

  @pl.loop(0, EROW)
  def _(i):
    for j in range(DIM // 16):
      fbufs[0][i, pl.ds(j * 16, 16)] = zero

  for k in range(ROWS_PER_TILE // EROW):
    pltpu.sync_copy(fbufs[0],
                    agg_sh.at[pl.ds(s * ROWS_PER_TILE + k * EROW, EROW)])
  rem = ROWS_PER_TILE % EROW
  if rem:
    pltpu.sync_copy(
        fbufs[0].at[pl.ds(0, rem)],
        agg_sh.at[pl.ds(s * ROWS_PER_TILE + ROWS_PER_TILE - rem, rem)])
  plsc.subcore_barrier()

  def meta_copies(i):
    sl = pl.ds(base + i * MCHUNK, MCHUNK)
    sv, dv = mbufs[i % 2]
    return ((src_hbm.at[sl], sv), (dst_hbm.at[sl], dv))

  def meta_start(i):
    for hbm, buf in meta_copies(i):
      pltpu.async_copy(hbm, buf, msem)

  def meta_wait(i):
    for hbm, buf in meta_copies(i):
      pltpu.make_async_copy(hbm, buf, msem).wait()

  meta_start(0)
  for i in range(N_MCHUNK):
    sv, dv = mbufs[i % 2]
    meta_wait(i)
    if i + 1 < N_MCHUNK:
      meta_start(i + 1)

    sl2 = pl.ds((base + i * MCHUNK) * 2, MCHUNK * 2)
    pltpu.sync_copy(w01_hbm.at[sl2], wv)
    pltpu.async_copy(pck_hbm.at[sv.at[0]], bbufs[0], gsems[0])
    pltpu.async_copy(pck_hbm.at[sv.at[1]], bbufs[1], gsems[1])

    @pl.loop(0, MCHUNK, step=2)
    def _(b):
      for p in range(2):
        r = b + p
        pltpu.make_async_copy(pck_hbm.at[sv.at[r]], bbufs[p],
                              gsems[p]).wait()
        if i > 0:
          pltpu.make_async_copy(fbufs[p], agg_sh.at[dv.at[r]],
                                ssems[p]).wait()
        else:
          @pl.when(r >= 2)
          def _():
            pltpu.make_async_copy(fbufs[p], agg_sh.at[dv.at[r]],
                                  ssems[p]).wait()

        @pl.loop(0, EROW)
        def _(e):
          ev = jnp.full((16,), e, jnp.int32)
          w0s = plsc.load_gather(wv, [jnp.full((16,), 2 * r, jnp.int32), ev])
          w1s = plsc.load_gather(
              wv, [jnp.full((16,), 2 * r + 1, jnp.int32), ev])
          for g in range(DIM // 16):
            y = bbufs[p][e, pl.ds(16 * g, 16)]
            flo = plsc.bitcast(lax.shift_left(y, 16), jnp.float32)
            fhi = plsc.bitcast(
                lax.bitwise_and(y, jnp.int32(-65536)), jnp.float32)
            fbufs[p][e, pl.ds(16 * g, 16)] = w0s * flo + w1s * fhi

        pltpu.async_copy(fbufs[p], agg_sh.at[dv.at[r]], ssems[p], add=True)

        @pl.when(r + 2 < MCHUNK)
        def _():
          pltpu.async_copy(pck_hbm.at[sv.at[r + 2]], bbufs[p], gsems[p])

    if i + 1 == N_MCHUNK:
      for p in range(2):
        pltpu.make_async_copy(fbufs[p], agg_sh.at[sv.at[0]], ssems[p]).wait()

  plsc.subcore_barrier()

  pltpu.sync_copy(agg_sh.at[pl.ds(s * ROWS_PER_TILE, ROWS_PER_TILE)],
                  out_hbm.at[c, pl.ds(s * ROWS_PER_TILE, ROWS_PER_TILE)])


def _sc_edge_agg(pck, src2d, dst2d, w01_2d):
  mesh = plsc.VectorSubcoreMesh(core_axis_name="c", subcore_axis_name="s",
                                num_cores=NC, num_subcores=NS)
  fn = pl.kernel(
      _sc_body,
      out_type=jax.ShapeDtypeStruct((NC, AGG_ROWS, DIM), jnp.float32),
      mesh=mesh,
      compiler_params=pltpu.CompilerParams(needs_layout_passes=False),
      scratch_types=[
          [[pltpu.VMEM((MCHUNK, EROW), jnp.int32),
            pltpu.VMEM((MCHUNK, EROW), jnp.int32)]] * 2,
          pltpu.VMEM((2 * MCHUNK, EROW), jnp.float32),
          [pltpu.VMEM((EROW, DIM), jnp.int32)] * 2,
          [pltpu.VMEM((EROW, DIM), jnp.float32)] * 2,
          pltpu.SemaphoreType.DMA,
          [pltpu.SemaphoreType.DMA] * 2,
          [pltpu.SemaphoreType.DMA] * 2,
          pltpu.VMEM_SHARED((AGG_ROWS, DIM), jnp.float32),
      ],
  )
  return fn(pck, src2d, dst2d, w01_2d)


def kernel(node_emb, edge_index, etypes, basis1, wcomp1, loop1, bias1,
           basis2, wcomp2, loop2, bias2):
  src = edge_index[0].astype(jnp.int32)
  dst = edge_index[1].astype(jnp.int32)
  et = etypes.astype(jnp.int32)

  pad = E_PAD - N_EDGES
  pad_iota = jnp.arange(pad, dtype=jnp.int32)
  src2d = jnp.concatenate([src, pad_iota % N_NODES]).reshape(IDX_ROWS, EROW)
  et2d = jnp.pad(et, (0, pad), constant_values=NUM_REL).reshape(
      E_PAD // DIM, DIM)
  dst2d = jnp.concatenate([dst, pad_iota % N_NODES]).reshape(IDX_ROWS, EROW)

  def wpack(w0, w1):
    return jnp.stack(
        [w0.reshape(IDX_ROWS, EROW), w1.reshape(IDX_ROWS, EROW)],
        axis=1).reshape(IDX_ROWS * 2, EROW)

  w01_l1 = wpack(*_wsel(et2d, wcomp1))
  w01_l2 = wpack(*_wsel(et2d, wcomp2))

  def layer(h, basis, w01, loop_w, bias, act):
    pck, hsl = _dense_pre(h, basis, loop_w, bias.reshape(1, DIM))
    parts = _sc_edge_agg(pck, src2d, dst2d, w01)
    return _combine(parts[0, :N_NODES], parts[1, :N_NODES], hsl, act)

  h1 = layer(node_emb, basis1, w01_l1, loop1, bias1, jnp.tanh)
  return layer(h1, basis2, w01_l2, loop2, bias2, jax.nn.relu)
